# baseline (speedup 1.0000x reference)
import jax
import jax.numpy as jnp
from jax import lax
from jax.experimental import pallas as pl
from jax.experimental.pallas import tpu as pltpu
from jax.experimental.pallas import tpu_sc as plsc

N_PROP = 5
NUM_GRAPHS = 128
N, E, D, DE = 10000, 320000, 128, 16

NC, NS, L = 2, 16, 16
NW = NC * NS
NPAD = 10240
EPAD = 327680
C = 40
EPT = EPAD // NW
NCH = EPT // C
NAGG = 10112
RPT = NAGG // NS
ZR = 40
TI_PAD = 10080

_mesh = plsc.VectorSubcoreMesh(
    core_axis_name="c", subcore_axis_name="s", num_cores=NC, num_subcores=NS)

_f32 = jnp.float32


def _edge_body(p_hbm, q_hbm, ce_hbm, fi_hbm, ti_hbm, out_hbm,
               pr0, qr0, cr0, pr1, qr1, cr1,
               if0, if1, if2, if3, it0, it1, it2, it3,
               aggr, sg0, sg1, si0, si1, si2, si3):
    c = lax.axis_index("c")
    s = lax.axis_index("s")
    wid = c * NS + s
    ebase = wid * EPT

    ifb = (if0, if1, if2, if3)
    itb = (it0, it1, it2, it3)
    bufs = ((pr0, qr0, cr0, sg0), (pr1, qr1, cr1, sg1))
    sib = (si0, si1, si2, si3)

    def idx_copies(j, slot):
        off = ebase + j * C
        return (
            pltpu.make_async_copy(fi_hbm.at[pl.ds(off, C)], ifb[slot], sib[slot]),
            pltpu.make_async_copy(ti_hbm.at[pl.ds(off, C)], itb[slot], sib[slot]),
        )

    def start_idx(j, slot):
        for d in idx_copies(j, slot):
            d.start()

    def gathers(j, slot, b):
        pr, qr, cr, sg = bufs[b]
        return (
            pltpu.make_async_copy(p_hbm.at[ifb[slot]], pr, sg),
            pltpu.make_async_copy(q_hbm.at[itb[slot]], qr, sg),
            pltpu.make_async_copy(
                ce_hbm.at[pl.ds((ebase + j * C) * D, C * D)], cr, sg),
        )

    def issue(j, slot, b):
        for d in idx_copies(j, slot):
            d.wait()
        for d in gathers(j, slot, b):
            d.start()

    def proc(j, slot, b):
        pr, qr, cr, _ = bufs[b]
        for d in gathers(j, slot, b):
            d.wait()

        def rowfn(i, carry):
            for g in range(D // L):
                sl = pl.ds(g * L, L)
                cf = cr[pl.ds(i * D + g * L, L)]
                pr[i, sl] = jnp.maximum(pr[i, sl] + qr[i, sl] + cf, 0.0)
            return carry
        lax.fori_loop(0, C, rowfn, 0)
        pltpu.sync_copy(pr, aggr.at[itb[slot]], add=True)

    for b in range(4):
        start_idx(b, b)

    def zrow(i, carry):
        for g in range(D // L):
            pr0[i, pl.ds(g * L, L)] = jnp.zeros((L,), _f32)
        return carry
    lax.fori_loop(0, ZR, zrow, 0)

    def zcp(i, carry):
        pltpu.sync_copy(pr0, aggr.at[pl.ds(s * RPT + i * ZR, ZR)])
        return carry
    lax.fori_loop(0, RPT // ZR, zcp, 0)
    pltpu.sync_copy(pr0, aggr.at[pl.ds(s * RPT + RPT - ZR, ZR)])
    plsc.subcore_barrier()

    issue(0, 0, 0)
    issue(1, 1, 1)

    def outer(it4, carry):
        for b in range(4):
            j = it4 * 4 + b
            proc(j, b, b % 2)

            @pl.when(j + 4 < NCH)
            def _():
                start_idx(j + 4, b)

            @pl.when(j + 2 < NCH)
            def _():
                issue(j + 2, (b + 2) % 4, b % 2)
        return carry
    lax.fori_loop(0, NCH // 4, outer, 0)

    plsc.subcore_barrier()
    pltpu.sync_copy(aggr.at[pl.ds(s * RPT, RPT)],
                    out_hbm.at[c, pl.ds(s * RPT, RPT)])


_edge_call = pl.kernel(
    _edge_body,
    out_type=jax.ShapeDtypeStruct((NC, NPAD, D), _f32),
    mesh=_mesh,
    scratch_types=[
        pltpu.VMEM((C, D), _f32),
        pltpu.VMEM((C, D), _f32),
        pltpu.VMEM((C * D,), _f32),
        pltpu.VMEM((C, D), _f32),
        pltpu.VMEM((C, D), _f32),
        pltpu.VMEM((C * D,), _f32),
        pltpu.VMEM((C,), jnp.int32),
        pltpu.VMEM((C,), jnp.int32),
        pltpu.VMEM((C,), jnp.int32),
        pltpu.VMEM((C,), jnp.int32),
        pltpu.VMEM((C,), jnp.int32),
        pltpu.VMEM((C,), jnp.int32),
        pltpu.VMEM((C,), jnp.int32),
        pltpu.VMEM((C,), jnp.int32),
        pltpu.VMEM_SHARED((NAGG, D), _f32),
        pltpu.SemaphoreType.DMA,
        pltpu.SemaphoreType.DMA,
        pltpu.SemaphoreType.DMA,
        pltpu.SemaphoreType.DMA,
        pltpu.SemaphoreType.DMA,
        pltpu.SemaphoreType.DMA,
    ],
)


def _deg_body(ti_hbm, out_hbm, idxt, ones_v, zbuf, deg_sh):
    c = lax.axis_index("c")
    s = lax.axis_index("s")
    wid = c * NS + s

    pltpu.sync_copy(ti_hbm.at[pl.ds(wid * (EPT // C), EPT // C)], idxt)

    def fill(i, carry):
        ones_v[i] = jnp.full((DE,), 1.0, _f32)
        return carry
    lax.fori_loop(0, C, fill, 0)

    def zrow(i, carry):
        zbuf[i] = jnp.zeros((DE,), _f32)
        return carry
    lax.fori_loop(0, ZR, zrow, 0)

    def zcp(i, carry):
        pltpu.sync_copy(zbuf, deg_sh.at[pl.ds(s * RPT + i * ZR, ZR)])
        return carry
    lax.fori_loop(0, RPT // ZR, zcp, 0)
    pltpu.sync_copy(zbuf, deg_sh.at[pl.ds(s * RPT + RPT - ZR, ZR)])
    plsc.subcore_barrier()

    def body(j, carry):
        pltpu.sync_copy(ones_v, deg_sh.at[idxt.at[j]], add=True)
        return carry
    lax.fori_loop(0, NCH, body, 0)

    plsc.subcore_barrier()
    pltpu.sync_copy(deg_sh.at[pl.ds(s * RPT, RPT)],
                    out_hbm.at[c, pl.ds(s * RPT, RPT)])


_deg_call = pl.kernel(
    _deg_body,
    out_type=jax.ShapeDtypeStruct((NC, NPAD, DE), _f32),
    mesh=_mesh,
    scratch_types=[
        pltpu.VMEM((NCH, C), jnp.int32),
        pltpu.VMEM((C, DE), _f32),
        pltpu.VMEM((ZR, DE), _f32),
        pltpu.VMEM_SHARED((NAGG, DE), _f32),
    ],
)


NB = 8
BN = NPAD // NB
BE = 4096


def _dot(a, b):
    return jnp.dot(a, b, preferred_element_type=_f32)


def _enc_body(nf, wne, bne, wf, wt, h_o, p_o, q_o):
    h = jnp.maximum(_dot(nf[...], wne[...]) + bne[...], 0.0)
    h_o[...] = h
    p_o[...] = _dot(h, wf[...])
    q_o[...] = _dot(h, wt[...])


def _enc_call(nf, wne, bne, wf, wt):
    blk = lambda r, cd: pl.BlockSpec((r, cd), lambda i: (i, 0))
    full = lambda r, cd: pl.BlockSpec((r, cd), lambda i: (0, 0))
    return pl.pallas_call(
        _enc_body,
        grid=(NB,),
        in_specs=[blk(BN, D), full(D, D), full(1, D), full(D, D), full(D, D)],
        out_specs=[blk(BN, D)] * 3,
        out_shape=[jax.ShapeDtypeStruct((NPAD, D), _f32)] * 3,
    )(nf, wne, bne, wf, wt)


def _ce_body(ef, wee, bee, we, bm1, ce_o):
    e = jnp.maximum(_dot(ef[...], wee[...]) + bee[...], 0.0)
    ce_o[...] = _dot(e, we[...]) + bm1[...]


def _ce_call(ef, wee, bee, we, bm1):
    blk = lambda r, cd: pl.BlockSpec((r, cd), lambda i: (i, 0))
    full = lambda r, cd: pl.BlockSpec((r, cd), lambda i: (0, 0))
    return pl.pallas_call(
        _ce_body,
        grid=(EPAD // BE,),
        in_specs=[blk(BE, DE), full(DE, DE), full(1, DE), full(DE, D), full(1, D)],
        out_specs=blk(BE, D),
        out_shape=jax.ShapeDtypeStruct((EPAD, D), _f32),
    )(ef, wee, bee, we, bm1)


def _upd_body(h, aa, ab, da, db, wm2, bm2, wu1a, wu1b, bu1, wu2, bu2, wf, wt,
              h_o, p_o, q_o):
    deg = da[0, :, 0:1] + db[0, :, 0:1]
    agg = _dot(aa[0] + ab[0], wm2[...]) + deg * bm2[...]
    u1 = jnp.maximum(_dot(h[...], wu1a[...]) + _dot(agg, wu1b[...]) + bu1[...], 0.0)
    hn = h[...] + _dot(u1, wu2[...]) + bu2[...]
    h_o[...] = hn
    p_o[...] = _dot(hn, wf[...])
    q_o[...] = _dot(hn, wt[...])


def _upd_call(h, aggr2, deg2, wm2, bm2, wu1a, wu1b, bu1, wu2, bu2, wf, wt):
    blk = lambda r, cd: pl.BlockSpec((r, cd), lambda i: (i, 0))
    full = lambda r, cd: pl.BlockSpec((r, cd), lambda i: (0, 0))
    a_spec = lambda cidx: pl.BlockSpec((1, BN, D), lambda i, _c=cidx: (_c, i, 0))
    d_spec = lambda cidx: pl.BlockSpec((1, BN, DE), lambda i, _c=cidx: (_c, i, 0))
    return pl.pallas_call(
        _upd_body,
        grid=(NB,),
        in_specs=[blk(BN, D), a_spec(0), a_spec(1), d_spec(0), d_spec(1),
                  full(D, D), full(1, D), full(D, 2 * D), full(D, 2 * D),
                  full(1, 2 * D), full(2 * D, D), full(1, D), full(D, D),
                  full(D, D)],
        out_specs=[blk(BN, D)] * 3,
        out_shape=[jax.ShapeDtypeStruct((NPAD, D), _f32)] * 3,
    )(h, aggr2, aggr2, deg2, deg2, wm2, bm2, wu1a, wu1b, bu1, wu2, bu2, wf, wt)


def _pool_body(gi, h, o):
    @pl.when(pl.program_id(0) == 0)
    def _():
        o[...] = jnp.zeros_like(o)
    oh = (gi[...] == lax.broadcasted_iota(jnp.int32, (1, NUM_GRAPHS), 1)
          ).astype(_f32)
    hm = jnp.where(gi[...] < NUM_GRAPHS, h[...], 0.0)
    o[...] += lax.dot_general(oh, hm, (((0,), (0,)), ((), ())),
                              preferred_element_type=_f32)


def _pool_call(gi, h):
    blk = lambda r, cd: pl.BlockSpec((r, cd), lambda i: (i, 0))
    return pl.pallas_call(
        _pool_body,
        grid=(NB,),
        in_specs=[blk(BN, 1), blk(BN, D)],
        out_specs=pl.BlockSpec((NUM_GRAPHS, D), lambda i: (0, 0)),
        out_shape=jax.ShapeDtypeStruct((NUM_GRAPHS, D), _f32),
    )(gi, h)


def kernel(node_features, edge_features, from_idx, to_idx, graph_idx,
           W_node_enc, b_node_enc, W_edge_enc, b_edge_enc,
           W_msg1, b_msg1, W_msg2, b_msg2,
           W_upd1, b_upd1, W_upd2, b_upd2):
    nf = jnp.pad(node_features, ((0, NPAD - N), (0, 0)))
    ef = jnp.pad(edge_features, ((0, EPAD - E), (0, 0)))
    fi = jnp.pad(from_idx.astype(jnp.int32), (0, EPAD - E))
    ti = jnp.pad(to_idx.astype(jnp.int32), (0, EPAD - E),
                 constant_values=jnp.int32(TI_PAD))
    gi = jnp.pad(graph_idx.astype(jnp.int32), (0, NPAD - N),
                 constant_values=jnp.int32(1 << 30))
    ti2 = ti.reshape(EPAD // C, C)
    gi2 = gi.reshape(NPAD, 1)

    Wf, Wt, We = W_msg1[:D], W_msg1[D:2 * D], W_msg1[2 * D:]
    Wu1a, Wu1b = W_upd1[:D], W_upd1[D:]
    row = lambda b: b.reshape(1, -1)

    h, P, Q = _enc_call(nf, W_node_enc, row(b_node_enc), Wf, Wt)
    ce = _ce_call(ef, W_edge_enc, row(b_edge_enc), We, row(b_msg1))
    ce_flat = ce.reshape(EPAD * D)
    deg2 = _deg_call(ti2)

    for _ in range(N_PROP):
        aggr2 = _edge_call(P, Q, ce_flat, fi, ti)
        h, P, Q = _upd_call(h, aggr2, deg2, W_msg2, row(b_msg2),
                            Wu1a, Wu1b, row(b_upd1), W_upd2, row(b_upd2),
                            Wf, Wt)

    return _pool_call(gi2, h)


# speedup vs baseline: 1.0297x; 1.0297x over previous
import jax
import jax.numpy as jnp
from jax import lax
from jax.experimental import pallas as pl
from jax.experimental.pallas import tpu as pltpu
from jax.experimental.pallas import tpu_sc as plsc

N_PROP = 5
NUM_GRAPHS = 128
N, E, D, DE = 10000, 320000, 128, 16

NC, NS, L = 2, 16, 16
NW = NC * NS
NPAD = 10240
EPAD = 327680
C = 40
EPT = EPAD // NW
NCH = EPT // C
NAGG = NPAD
RPT = NAGG // NS
ZR = 32
TI_PAD = NPAD - 8

_mesh = plsc.VectorSubcoreMesh(
    core_axis_name="c", subcore_axis_name="s", num_cores=NC, num_subcores=NS)

_f32 = jnp.float32


def _edge_body(p_hbm, q_hbm, ce_hbm, fi_hbm, ti_hbm, out_hbm,
               pr0, qr0, cr0, pr1, qr1, cr1,
               if0, if1, if2, if3, it0, it1, it2, it3,
               zbuf, aggr, sg0, sg1, si0, si1, si2, si3):
    c = lax.axis_index("c")
    s = lax.axis_index("s")
    wid = c * NS + s
    ebase = wid * EPT

    ifb = (if0, if1, if2, if3)
    itb = (it0, it1, it2, it3)
    bufs = ((pr0, qr0, cr0, sg0), (pr1, qr1, cr1, sg1))
    sib = (si0, si1, si2, si3)

    def idx_copies(j, slot):
        off = ebase + j * C
        return (
            pltpu.make_async_copy(fi_hbm.at[pl.ds(off, C)], ifb[slot], sib[slot]),
            pltpu.make_async_copy(ti_hbm.at[pl.ds(off, C)], itb[slot], sib[slot]),
        )

    def start_idx(j, slot):
        for d in idx_copies(j, slot):
            d.start()

    def gathers(j, slot, b):
        pr, qr, cr, sg = bufs[b]
        return (
            pltpu.make_async_copy(p_hbm.at[ifb[slot]], pr, sg),
            pltpu.make_async_copy(q_hbm.at[itb[slot]], qr, sg),
            pltpu.make_async_copy(ce_hbm.at[pl.ds(ebase + j * C, C)], cr, sg),
        )

    def issue(j, slot, b):
        for d in idx_copies(j, slot):
            d.wait()
        for d in gathers(j, slot, b):
            d.start()

    def proc(j, slot, b):
        pr, qr, cr, _ = bufs[b]
        for d in gathers(j, slot, b):
            d.wait()

        def rowfn(i, carry):
            for g in range(D // L):
                sl = pl.ds(g * L, L)
                pr[i, sl] = jnp.maximum(pr[i, sl] + qr[i, sl] + cr[i, sl], 0.0)
            return carry
        lax.fori_loop(0, C, rowfn, 0)
        pltpu.sync_copy(pr, aggr.at[itb[slot]], add=True)

    for b in range(4):
        start_idx(b, b)

    def zrow(i, carry):
        for g in range(D // L):
            zbuf[i, pl.ds(g * L, L)] = jnp.zeros((L,), _f32)
        return carry
    lax.fori_loop(0, ZR, zrow, 0)

    def zcp(i, carry):
        pltpu.sync_copy(zbuf, aggr.at[pl.ds(s * RPT + i * ZR, ZR)])
        return carry
    lax.fori_loop(0, RPT // ZR, zcp, 0)
    plsc.subcore_barrier()

    issue(0, 0, 0)
    issue(1, 1, 1)

    def outer(it4, carry):
        for b in range(4):
            j = it4 * 4 + b
            proc(j, b, b % 2)

            @pl.when(j + 4 < NCH)
            def _():
                start_idx(j + 4, b)

            @pl.when(j + 2 < NCH)
            def _():
                issue(j + 2, (b + 2) % 4, b % 2)
        return carry
    lax.fori_loop(0, NCH // 4, outer, 0)

    plsc.subcore_barrier()
    pltpu.sync_copy(aggr.at[pl.ds(s * RPT, RPT)],
                    out_hbm.at[c, pl.ds(s * RPT, RPT)])


_edge_call = pl.kernel(
    _edge_body,
    out_type=jax.ShapeDtypeStruct((NC, NPAD, D), _f32),
    mesh=_mesh,
    scratch_types=[
        pltpu.VMEM((C, D), _f32),
        pltpu.VMEM((C, D), _f32),
        pltpu.VMEM((C, D), _f32),
        pltpu.VMEM((C, D), _f32),
        pltpu.VMEM((C, D), _f32),
        pltpu.VMEM((C, D), _f32),
        pltpu.VMEM((C,), jnp.int32),
        pltpu.VMEM((C,), jnp.int32),
        pltpu.VMEM((C,), jnp.int32),
        pltpu.VMEM((C,), jnp.int32),
        pltpu.VMEM((C,), jnp.int32),
        pltpu.VMEM((C,), jnp.int32),
        pltpu.VMEM((C,), jnp.int32),
        pltpu.VMEM((C,), jnp.int32),
        pltpu.VMEM((ZR, D), _f32),
        pltpu.VMEM_SHARED((NAGG, D), _f32),
        pltpu.SemaphoreType.DMA,
        pltpu.SemaphoreType.DMA,
        pltpu.SemaphoreType.DMA,
        pltpu.SemaphoreType.DMA,
        pltpu.SemaphoreType.DMA,
        pltpu.SemaphoreType.DMA,
    ],
)


def _deg_body(ti_hbm, out_hbm, idxt, ones_v, zbuf, deg_sh):
    c = lax.axis_index("c")
    s = lax.axis_index("s")
    wid = c * NS + s

    pltpu.sync_copy(ti_hbm.at[pl.ds(wid * (EPT // C), EPT // C)], idxt)

    def fill(i, carry):
        ones_v[i] = jnp.full((DE,), 1.0, _f32)
        return carry
    lax.fori_loop(0, C, fill, 0)

    def zrow(i, carry):
        zbuf[i] = jnp.zeros((DE,), _f32)
        return carry
    lax.fori_loop(0, ZR, zrow, 0)

    def zcp(i, carry):
        pltpu.sync_copy(zbuf, deg_sh.at[pl.ds(s * RPT + i * ZR, ZR)])
        return carry
    lax.fori_loop(0, RPT // ZR, zcp, 0)
    plsc.subcore_barrier()

    def body(j, carry):
        pltpu.sync_copy(ones_v, deg_sh.at[idxt.at[j]], add=True)
        return carry
    lax.fori_loop(0, NCH, body, 0)

    plsc.subcore_barrier()
    pltpu.sync_copy(deg_sh.at[pl.ds(s * RPT, RPT)],
                    out_hbm.at[c, pl.ds(s * RPT, RPT)])


_deg_call = pl.kernel(
    _deg_body,
    out_type=jax.ShapeDtypeStruct((NC, NPAD, DE), _f32),
    mesh=_mesh,
    scratch_types=[
        pltpu.VMEM((NCH, C), jnp.int32),
        pltpu.VMEM((C, DE), _f32),
        pltpu.VMEM((ZR, DE), _f32),
        pltpu.VMEM_SHARED((NAGG, DE), _f32),
    ],
)


NB = 8
BN = NPAD // NB
BE = 4096


def _dot(a, b):
    return jnp.dot(a, b, preferred_element_type=_f32)


def _enc_body(nf, wne, bne, wf, wt, h_o, p_o, q_o):
    h = jnp.maximum(_dot(nf[...], wne[...]) + bne[...], 0.0)
    h_o[...] = h
    p_o[...] = _dot(h, wf[...])
    q_o[...] = _dot(h, wt[...])


def _enc_call(nf, wne, bne, wf, wt):
    blk = lambda r, cd: pl.BlockSpec((r, cd), lambda i: (i, 0))
    full = lambda r, cd: pl.BlockSpec((r, cd), lambda i: (0, 0))
    return pl.pallas_call(
        _enc_body,
        grid=(NB,),
        in_specs=[blk(BN, D), full(D, D), full(1, D), full(D, D), full(D, D)],
        out_specs=[blk(BN, D)] * 3,
        out_shape=[jax.ShapeDtypeStruct((NPAD, D), _f32)] * 3,
    )(nf, wne, bne, wf, wt)


def _ce_body(ef, wee, bee, we, bm1, ce_o):
    e = jnp.maximum(_dot(ef[...], wee[...]) + bee[...], 0.0)
    ce_o[...] = _dot(e, we[...]) + bm1[...]


def _ce_call(ef, wee, bee, we, bm1):
    blk = lambda r, cd: pl.BlockSpec((r, cd), lambda i: (i, 0))
    full = lambda r, cd: pl.BlockSpec((r, cd), lambda i: (0, 0))
    return pl.pallas_call(
        _ce_body,
        grid=(EPAD // BE,),
        in_specs=[blk(BE, DE), full(DE, DE), full(1, DE), full(DE, D), full(1, D)],
        out_specs=blk(BE, D),
        out_shape=jax.ShapeDtypeStruct((EPAD, D), _f32),
    )(ef, wee, bee, we, bm1)


def _upd_body(h, aa, ab, da, db, wm2, bm2, wu1a, wu1b, bu1, wu2, bu2, wf, wt,
              h_o, p_o, q_o):
    deg = da[0, :, 0:1] + db[0, :, 0:1]
    agg = _dot(aa[0] + ab[0], wm2[...]) + deg * bm2[...]
    u1 = jnp.maximum(_dot(h[...], wu1a[...]) + _dot(agg, wu1b[...]) + bu1[...], 0.0)
    hn = h[...] + _dot(u1, wu2[...]) + bu2[...]
    h_o[...] = hn
    p_o[...] = _dot(hn, wf[...])
    q_o[...] = _dot(hn, wt[...])


def _upd_call(h, aggr2, deg2, wm2, bm2, wu1a, wu1b, bu1, wu2, bu2, wf, wt):
    blk = lambda r, cd: pl.BlockSpec((r, cd), lambda i: (i, 0))
    full = lambda r, cd: pl.BlockSpec((r, cd), lambda i: (0, 0))
    a_spec = lambda cidx: pl.BlockSpec((1, BN, D), lambda i, _c=cidx: (_c, i, 0))
    d_spec = lambda cidx: pl.BlockSpec((1, BN, DE), lambda i, _c=cidx: (_c, i, 0))
    return pl.pallas_call(
        _upd_body,
        grid=(NB,),
        in_specs=[blk(BN, D), a_spec(0), a_spec(1), d_spec(0), d_spec(1),
                  full(D, D), full(1, D), full(D, 2 * D), full(D, 2 * D),
                  full(1, 2 * D), full(2 * D, D), full(1, D), full(D, D),
                  full(D, D)],
        out_specs=[blk(BN, D)] * 3,
        out_shape=[jax.ShapeDtypeStruct((NPAD, D), _f32)] * 3,
    )(h, aggr2, aggr2, deg2, deg2, wm2, bm2, wu1a, wu1b, bu1, wu2, bu2, wf, wt)


def _pool_body(gi, h, o):
    @pl.when(pl.program_id(0) == 0)
    def _():
        o[...] = jnp.zeros_like(o)
    oh = (gi[...] == lax.broadcasted_iota(jnp.int32, (1, NUM_GRAPHS), 1)
          ).astype(_f32)
    hm = jnp.where(gi[...] < NUM_GRAPHS, h[...], 0.0)
    o[...] += lax.dot_general(oh, hm, (((0,), (0,)), ((), ())),
                              preferred_element_type=_f32)


def _pool_call(gi, h):
    blk = lambda r, cd: pl.BlockSpec((r, cd), lambda i: (i, 0))
    return pl.pallas_call(
        _pool_body,
        grid=(NB,),
        in_specs=[blk(BN, 1), blk(BN, D)],
        out_specs=pl.BlockSpec((NUM_GRAPHS, D), lambda i: (0, 0)),
        out_shape=jax.ShapeDtypeStruct((NUM_GRAPHS, D), _f32),
    )(gi, h)


def kernel(node_features, edge_features, from_idx, to_idx, graph_idx,
           W_node_enc, b_node_enc, W_edge_enc, b_edge_enc,
           W_msg1, b_msg1, W_msg2, b_msg2,
           W_upd1, b_upd1, W_upd2, b_upd2):
    nf = jnp.pad(node_features, ((0, NPAD - N), (0, 0)))
    ef = jnp.pad(edge_features, ((0, EPAD - E), (0, 0)))
    fi = jnp.pad(from_idx.astype(jnp.int32), (0, EPAD - E))
    ti = jnp.pad(to_idx.astype(jnp.int32), (0, EPAD - E),
                 constant_values=jnp.int32(TI_PAD))
    gi = jnp.pad(graph_idx.astype(jnp.int32), (0, NPAD - N),
                 constant_values=jnp.int32(1 << 30))
    ti2 = ti.reshape(EPAD // C, C)
    gi2 = gi.reshape(NPAD, 1)

    Wf, Wt, We = W_msg1[:D], W_msg1[D:2 * D], W_msg1[2 * D:]
    Wu1a, Wu1b = W_upd1[:D], W_upd1[D:]
    row = lambda b: b.reshape(1, -1)

    h, P, Q = _enc_call(nf, W_node_enc, row(b_node_enc), Wf, Wt)
    ce = _ce_call(ef, W_edge_enc, row(b_edge_enc), We, row(b_msg1))
    deg2 = _deg_call(ti2)

    for _ in range(N_PROP):
        aggr2 = _edge_call(P, Q, ce, fi, ti)
        h, P, Q = _upd_call(h, aggr2, deg2, W_msg2, row(b_msg2),
                            Wu1a, Wu1b, row(b_upd1), W_upd2, row(b_upd2),
                            Wf, Wt)

    return _pool_call(gi2, h)
